# baseline (speedup 1.0000x reference)
import functools

import jax
import jax.numpy as jnp
from jax import lax
from jax.experimental import pallas as pl
from jax.experimental.pallas import tpu as pltpu
from jax.experimental.pallas import tpu_sc as plsc

NUM_CORES = 2
NUM_SUBCORES = 16
NUM_WORKERS = NUM_CORES * NUM_SUBCORES
LANES = 16

BATCH = 16384
DIM = 64
SUB = 8
B_PER_W = BATCH // NUM_WORKERS
CHUNK = 64
N_CHUNKS = B_PER_W // CHUNK
GPC = CHUNK // LANES


def _mf_body(user_hbm, item_hbm, uemb_hbm, iemb_hbm, ubias_hbm, ibias_hbm,
             gbias_hbm, out_hbm,
             uidx_v, iidx_v, ur0, ir0, ur1, ir1, ub0, ib0, ub1, ib1,
             out_v, part_v, gb_v, sem0, sem1):
    wid = lax.axis_index("s") * NUM_CORES + lax.axis_index("c")
    base = wid * B_PER_W

    pltpu.sync_copy(user_hbm.at[pl.ds(base, B_PER_W)], uidx_v)
    pltpu.sync_copy(item_hbm.at[pl.ds(base, B_PER_W)], iidx_v)
    pltpu.sync_copy(gbias_hbm, gb_v.at[pl.ds(0, 1)])

    bufs = [(ur0, ir0, ub0, ib0, sem0), (ur1, ir1, ub1, ib1, sem1)]

    def row_src(emb, idx):
        blk = pl.multiple_of((idx // SUB) * SUB, SUB)
        return emb.at[pl.ds(blk, SUB), :].at[pl.ds(idx % SUB, 1), :]

    def fire_chunk(c, parity):
        urb, irb, ubb, ibb, sem = bufs[parity]

        def body(g, carry):
            u_vec = uidx_v[pl.ds(c * CHUNK + g * LANES, LANES)]
            i_vec = iidx_v[pl.ds(c * CHUNK + g * LANES, LANES)]
            for r in range(LANES):
                j = g * LANES + r
                pltpu.async_copy(row_src(uemb_hbm, u_vec[r]),
                                 urb.at[pl.ds(j, 1), :], sem)
                pltpu.async_copy(row_src(iemb_hbm, i_vec[r]),
                                 irb.at[pl.ds(j, 1), :], sem)
                pltpu.async_copy(row_src(ubias_hbm, u_vec[r]),
                                 ubb.at[pl.ds(j, 1), :], sem)
                pltpu.async_copy(row_src(ibias_hbm, i_vec[r]),
                                 ibb.at[pl.ds(j, 1), :], sem)
            return carry

        lax.fori_loop(0, GPC, body, 0)

    def drain_chunk(parity):
        urb, irb, ubb, ibb, sem = bufs[parity]
        pltpu.make_async_copy(uemb_hbm.at[pl.ds(0, CHUNK), :], urb,
                              sem).wait()
        pltpu.make_async_copy(iemb_hbm.at[pl.ds(0, CHUNK), :], irb,
                              sem).wait()
        pltpu.make_async_copy(ubias_hbm.at[pl.ds(0, CHUNK), :], ubb,
                              sem).wait()
        pltpu.make_async_copy(ibias_hbm.at[pl.ds(0, CHUNK), :], ibb,
                              sem).wait()

    lane_iota = lax.iota(jnp.int32, LANES)
    zero_idx = lane_iota * 0

    def compute_chunk(c, parity):
        urb, irb, ubb, ibb, _ = bufs[parity]
        gb = gb_v[...][0]

        def group_body(g, carry):
            for r16 in range(LANES):
                urow = urb.at[g * LANES + r16]
                irow = irb.at[g * LANES + r16]
                s = urow[pl.ds(0, LANES)] * irow[pl.ds(0, LANES)]
                for cc in range(1, DIM // LANES):
                    s = s + (urow[pl.ds(cc * LANES, LANES)]
                             * irow[pl.ds(cc * LANES, LANES)])
                part_v[pl.ds(r16 * LANES, LANES)] = s
            row_idx = lane_iota + g * LANES
            bu = plsc.load_gather(ubb, [row_idx, zero_idx])
            bi = plsc.load_gather(ibb, [row_idx, zero_idx])
            acc = bu + bi + gb
            for cc in range(LANES):
                acc = acc + plsc.load_gather(part_v, [lane_iota * LANES + cc])
            out_v[pl.ds(c * CHUNK + g * LANES, LANES)] = acc
            return carry

        lax.fori_loop(0, GPC, group_body, 0)

    fire_chunk(0, 0)
    fire_chunk(1, 1)

    def pipeline_body(k, carry):
        c0 = 2 * k
        drain_chunk(0)
        compute_chunk(c0, 0)
        fire_chunk(c0 + 2, 0)
        drain_chunk(1)
        compute_chunk(c0 + 1, 1)
        fire_chunk(c0 + 3, 1)
        return carry

    lax.fori_loop(0, N_CHUNKS // 2 - 1, pipeline_body, 0)
    drain_chunk(0)
    compute_chunk(N_CHUNKS - 2, 0)
    drain_chunk(1)
    compute_chunk(N_CHUNKS - 1, 1)

    pltpu.sync_copy(out_v, out_hbm.at[pl.ds(base, B_PER_W)])


_mf_kernel = functools.partial(
    pl.kernel,
    out_type=jax.ShapeDtypeStruct((BATCH,), jnp.float32),
    mesh=plsc.VectorSubcoreMesh(core_axis_name="c", subcore_axis_name="s",
                                num_cores=NUM_CORES,
                                num_subcores=NUM_SUBCORES),
    scratch_types=[
        pltpu.VMEM((B_PER_W,), jnp.int32),
        pltpu.VMEM((B_PER_W,), jnp.int32),
        pltpu.VMEM((CHUNK, DIM), jnp.float32),
        pltpu.VMEM((CHUNK, DIM), jnp.float32),
        pltpu.VMEM((CHUNK, DIM), jnp.float32),
        pltpu.VMEM((CHUNK, DIM), jnp.float32),
        pltpu.VMEM((CHUNK, 1), jnp.float32),
        pltpu.VMEM((CHUNK, 1), jnp.float32),
        pltpu.VMEM((CHUNK, 1), jnp.float32),
        pltpu.VMEM((CHUNK, 1), jnp.float32),
        pltpu.VMEM((B_PER_W,), jnp.float32),
        pltpu.VMEM((LANES * LANES,), jnp.float32),
        pltpu.VMEM((LANES,), jnp.float32),
        pltpu.SemaphoreType.DMA,
        pltpu.SemaphoreType.DMA,
    ],
    compiler_params=pltpu.CompilerParams(needs_layout_passes=False,
                                         use_tc_tiling_on_sc=True),
)(_mf_body)


@jax.jit
def kernel(user, item, user_emb, item_emb, user_bias, item_bias, global_bias):
    user = user.astype(jnp.int32)
    item = item.astype(jnp.int32)
    return _mf_kernel(user, item, user_emb, item_emb,
                      user_bias, item_bias, global_bias)
